# baseline (speedup 1.0000x reference)
import functools

import jax
import jax.numpy as jnp
from jax import lax
from jax.experimental import pallas as pl
from jax.experimental.pallas import tpu as pltpu
from jax.experimental.pallas import tpu_sc as plsc

N = 10000
NPAD = 10240
D_IN = 128
HIDDEN = 128
OUT = 64
E = 160000
CH = 128
NC = 2
NS = 16
NW = NC * NS
E_PAD = 163840
CPW = E_PAD // (NW * CH)
EPW = CPW * CH
RPT = NPAD // NS


def _zero16():
    return jnp.zeros((16,), jnp.float32)

def _worker_id():
    return lax.axis_index("s") * NC + lax.axis_index("c")


@functools.partial(
    pl.kernel,
    out_type=jax.ShapeDtypeStruct((NW, 6, NPAD), jnp.float32),
    mesh=plsc.VectorSubcoreMesh(core_axis_name="c", subcore_axis_name="s"),
    scratch_types=[
        [pltpu.VMEM((EPW,), jnp.int32) for _ in range(6)],
        [pltpu.VMEM((NPAD,), jnp.float32) for _ in range(6)],
        pltpu.SemaphoreType.DMA,
    ],
    compiler_params=pltpu.CompilerParams(needs_layout_passes=False),
)
def _degrees_sc(e0, e1, e2, e3, e4, e5, out, idxs, hists, sem):
    wid = _worker_id()
    base = wid * EPW

    copies = [
        pltpu.async_copy(arr.at[pl.ds(base, EPW)], idx_v, sem)
        for arr, idx_v in zip((e0, e1, e2, e3, e4, e5), idxs)
    ]

    zero16 = _zero16()
    one16 = jnp.ones((16,), jnp.float32)

    def zbody(i, _):
        for h in hists:
            h[pl.ds(i * 16, 16)] = zero16
        return 0
    lax.fori_loop(0, NPAD // 16, zbody, 0)
    for cp in copies:
        cp.wait()

    for idx_v, hist in zip(idxs, hists):
        def body(i, _, idx_v=idx_v, hist=hist):
            idx16 = idx_v[pl.ds(i * 16, 16)]
            plsc.addupdate_scatter(hist, [idx16], one16)
            return 0
        lax.fori_loop(0, EPW // 16, body, 0)

    for r, hist in enumerate(hists):
        pltpu.sync_copy(hist, out.at[wid, r])


def _make_agg(D):
    @functools.partial(
        pl.kernel,
        out_type=jax.ShapeDtypeStruct((NC, NPAD, D), jnp.float32),
        mesh=plsc.VectorSubcoreMesh(core_axis_name="c", subcore_axis_name="s"),
        scratch_types=[
            pltpu.VMEM((CPW, CH), jnp.int32),
            pltpu.VMEM((CPW, CH), jnp.int32),
            pltpu.VMEM((CH, D), jnp.float32),
            pltpu.VMEM((CH, D), jnp.float32),
            pltpu.VMEM_SHARED((NPAD, D), jnp.float32),
            pltpu.SemaphoreType.DMA,
            pltpu.SemaphoreType.DMA,
        ],
        compiler_params=pltpu.CompilerParams(use_tc_tiling_on_sc=False),
    )
    def agg(y, src, dst, out, sidx, didx, rows0, rows1, acc_sh,
            sem_i, sem_g):
        c = lax.axis_index("c")
        s = lax.axis_index("s")
        wid = s * NC + c

        di0 = pltpu.async_copy(src.at[pl.ds(wid * CPW, CPW)], sidx, sem_i)
        di1 = pltpu.async_copy(dst.at[pl.ds(wid * CPW, CPW)], didx, sem_i)

        zero16 = _zero16()

        def zbody(i, _):
            for j in range(D // 16):
                rows0[i, pl.ds(j * 16, 16)] = zero16
            return 0
        lax.fori_loop(0, CH, zbody, 0)
        for q in range(RPT // CH):
            pltpu.sync_copy(rows0, acc_sh.at[pl.ds(s * RPT + q * CH, CH)])
        di0.wait()
        di1.wait()
        plsc.subcore_barrier()

        def body(t, _):
            pltpu.async_copy(y.at[sidx.at[t]], rows0, sem_g).wait()
            pltpu.sync_copy(rows0, acc_sh.at[didx.at[t]], add=True)
            return 0
        lax.fori_loop(0, CPW, body, 0)
        plsc.subcore_barrier()

        for q in range(RPT // CH):
            off = s * RPT + q * CH
            pltpu.sync_copy(acc_sh.at[pl.ds(off, CH)], rows0)
            pltpu.sync_copy(rows0, out.at[c, pl.ds(off, CH)])

    return agg


_agg_h = _make_agg(HIDDEN)
_agg_o = _make_agg(OUT)


def _degsum_body(dp_ref, out_ref):
    dg = jnp.sum(dp_ref[...], axis=0)
    out_ref[...] = jnp.where(dg > 0, lax.rsqrt(jnp.maximum(dg, 1.0)), 0.0)


def _norms_tc(degparts):
    return pl.pallas_call(
        _degsum_body,
        grid=(NPAD // 128,),
        in_specs=[pl.BlockSpec((NW, 6, 128), lambda i: (0, 0, i))],
        out_specs=pl.BlockSpec((6, 128), lambda i: (0, i)),
        out_shape=jax.ShapeDtypeStruct((6, NPAD), jnp.float32),
    )(degparts)


def _proj_body(x_ref, n_ref, w_ref, o_ref):
    o_ref[...] = jnp.dot(x_ref[...] * n_ref[...], w_ref[...],
                         preferred_element_type=jnp.float32)


def _proj_tc(x, ncol, W):
    H = W.shape[1]
    return pl.pallas_call(
        _proj_body,
        grid=(NPAD // 128,),
        in_specs=[
            pl.BlockSpec((128, 128), lambda i: (i, 0)),
            pl.BlockSpec((128, 1), lambda i: (i, 0)),
            pl.BlockSpec((128, H), lambda i: (0, 0)),
        ],
        out_specs=pl.BlockSpec((128, H), lambda i: (i, 0)),
        out_shape=jax.ShapeDtypeStruct((NPAD, H), jnp.float32),
    )(x, ncol, W)


def _comb2_body(relu, a_ref, c_ref, na_ref, nc_ref, ba_ref, bc_ref, o_ref):
    v = ((a_ref[0] + a_ref[1]) * na_ref[...]
         + (c_ref[0] + c_ref[1]) * nc_ref[...]
         + ba_ref[...] + bc_ref[...])
    o_ref[...] = jnp.maximum(v, 0.0) if relu else v


def _comb2_tc(agg_a, agg_c, n_a, n_c, b_a, b_c, relu):
    D = agg_a.shape[-1]
    return pl.pallas_call(
        functools.partial(_comb2_body, relu),
        grid=(NPAD // 128,),
        in_specs=[
            pl.BlockSpec((NC, 128, D), lambda i: (0, i, 0)),
            pl.BlockSpec((NC, 128, D), lambda i: (0, i, 0)),
            pl.BlockSpec((128, 1), lambda i: (i, 0)),
            pl.BlockSpec((128, 1), lambda i: (i, 0)),
            pl.BlockSpec((1, D), lambda i: (0, 0)),
            pl.BlockSpec((1, D), lambda i: (0, 0)),
        ],
        out_specs=pl.BlockSpec((128, D), lambda i: (i, 0)),
        out_shape=jax.ShapeDtypeStruct((NPAD, D), jnp.float32),
    )(agg_a, agg_c, n_a, n_c, b_a.reshape(1, D), b_c.reshape(1, D))


def _comb1_body(relu, a_ref, na_ref, ba_ref, o_ref):
    v = (a_ref[0] + a_ref[1]) * na_ref[...] + ba_ref[...]
    o_ref[...] = jnp.maximum(v, 0.0) if relu else v


def _comb1_tc(agg_a, n_a, b_a, relu):
    D = agg_a.shape[-1]
    return pl.pallas_call(
        functools.partial(_comb1_body, relu),
        grid=(NPAD // 128,),
        in_specs=[
            pl.BlockSpec((NC, 128, D), lambda i: (0, i, 0)),
            pl.BlockSpec((128, 1), lambda i: (i, 0)),
            pl.BlockSpec((1, D), lambda i: (0, 0)),
        ],
        out_specs=pl.BlockSpec((128, D), lambda i: (i, 0)),
        out_shape=jax.ShapeDtypeStruct((NPAD, D), jnp.float32),
    )(agg_a, n_a, b_a.reshape(1, D))


def kernel(x_paper, x_author, edge_writes, edge_cites, edge_written_by,
           W1_writes, b1_writes, W1_cites, b1_cites, W1_written_by, b1_written_by,
           W2_writes, b2_writes, W2_cites, b2_cites, W2_written_by, b2_written_by):
    pad = ((0, NPAD - N), (0, 0))
    xp = jnp.pad(x_paper, pad)
    xa = jnp.pad(x_author, pad)

    def _epad(e):
        return jnp.pad(e, ((0, 0), (0, E_PAD - E)), constant_values=NPAD - 1)

    ew = _epad(edge_writes)
    ec = _epad(edge_cites)
    eb = _epad(edge_written_by)
    ew_s, ew_d = ew[0], ew[1]
    ec_s, ec_d = ec[0], ec[1]
    eb_s, eb_d = eb[0], eb[1]
    ew_s2, ew_d2 = ew_s.reshape(-1, CH), ew_d.reshape(-1, CH)
    ec_s2, ec_d2 = ec_s.reshape(-1, CH), ec_d.reshape(-1, CH)
    eb_s2, eb_d2 = eb_s.reshape(-1, CH), eb_d.reshape(-1, CH)

    degparts = _degrees_sc(ew_s, ew_d, ec_s, ec_d, eb_s, eb_d)
    norms = _norms_tc(degparts)
    nsw = norms[0].reshape(NPAD, 1)
    ndw = norms[1].reshape(NPAD, 1)
    nsc = norms[2].reshape(NPAD, 1)
    ndc = norms[3].reshape(NPAD, 1)
    nsb = norms[4].reshape(NPAD, 1)
    ndb = norms[5].reshape(NPAD, 1)

    y1a = _proj_tc(xa, nsw, W1_writes)
    y1c = _proj_tc(xp, nsc, W1_cites)
    y1b = _proj_tc(xp, nsb, W1_written_by)
    s1w = _agg_h(y1a, ew_s2, ew_d2)
    s1c = _agg_h(y1c, ec_s2, ec_d2)
    s1b = _agg_h(y1b, eb_s2, eb_d2)
    h_paper = _comb2_tc(s1w, s1c, ndw, ndc, b1_writes, b1_cites, relu=True)
    h_author = _comb1_tc(s1b, ndb, b1_written_by, relu=True)

    y2a = _proj_tc(h_author, nsw, W2_writes)
    y2c = _proj_tc(h_paper, nsc, W2_cites)
    y2b = _proj_tc(h_paper, nsb, W2_written_by)
    s2w = _agg_o(y2a, ew_s2, ew_d2)
    s2c = _agg_o(y2c, ec_s2, ec_d2)
    s2b = _agg_o(y2b, eb_s2, eb_d2)
    out_paper = _comb2_tc(s2w, s2c, ndw, ndc, b2_writes, b2_cites, relu=False)
    out_author = _comb1_tc(s2b, ndb, b2_written_by, relu=False)

    return out_paper[:N], out_author[:N]


# speedup vs baseline: 2.2762x; 2.2762x over previous
import functools

import jax
import jax.numpy as jnp
from jax import lax
from jax.experimental import pallas as pl
from jax.experimental.pallas import tpu as pltpu
from jax.experimental.pallas import tpu_sc as plsc

N = 10000
NPAD = 10240
D_IN = 128
HIDDEN = 128
OUT = 64
E = 160000
CH = 128
NC = 2
NS = 16
NW = NC * NS
E_PAD = 163840
CPW = E_PAD // (NW * CH)
EPW = CPW * CH
RPT = NPAD // NS


def _zero16():
    return jnp.zeros((16,), jnp.float32)

def _worker_id():
    return lax.axis_index("s") * NC + lax.axis_index("c")


@functools.partial(
    pl.kernel,
    out_type=jax.ShapeDtypeStruct((NW, 6, NPAD), jnp.float32),
    mesh=plsc.VectorSubcoreMesh(core_axis_name="c", subcore_axis_name="s"),
    scratch_types=[
        [pltpu.VMEM((EPW,), jnp.int32) for _ in range(6)],
        [pltpu.VMEM((NPAD,), jnp.float32) for _ in range(6)],
        pltpu.SemaphoreType.DMA,
    ],
    compiler_params=pltpu.CompilerParams(needs_layout_passes=False),
)
def _degrees_sc(e0, e1, e2, e3, e4, e5, out, idxs, hists, sem):
    wid = _worker_id()
    base = wid * EPW

    copies = [
        pltpu.async_copy(arr.at[pl.ds(base, EPW)], idx_v, sem)
        for arr, idx_v in zip((e0, e1, e2, e3, e4, e5), idxs)
    ]

    zero16 = _zero16()
    one16 = jnp.ones((16,), jnp.float32)

    def zbody(i, _):
        for h in hists:
            h[pl.ds(i * 16, 16)] = zero16
        return 0
    lax.fori_loop(0, NPAD // 16, zbody, 0)
    for cp in copies:
        cp.wait()

    for idx_v, hist in zip(idxs, hists):
        def body(i, _, idx_v=idx_v, hist=hist):
            idx16 = idx_v[pl.ds(i * 16, 16)]
            plsc.addupdate_scatter(hist, [idx16], one16)
            return 0
        lax.fori_loop(0, EPW // 16, body, 0)

    for r, hist in enumerate(hists):
        pltpu.sync_copy(hist, out.at[wid, r])


def _make_agg(D):
    @functools.partial(
        pl.kernel,
        out_type=jax.ShapeDtypeStruct((NC, NPAD, D), jnp.float32),
        mesh=plsc.VectorSubcoreMesh(core_axis_name="c", subcore_axis_name="s"),
        scratch_types=[
            pltpu.VMEM((CPW, CH), jnp.int32),
            pltpu.VMEM((CPW, CH), jnp.int32),
            pltpu.VMEM((CH, D), jnp.float32),
            pltpu.VMEM((CH, D), jnp.float32),
            pltpu.VMEM_SHARED((NPAD, D), jnp.float32),
            pltpu.SemaphoreType.DMA,
            pltpu.SemaphoreType.DMA,
        ],
        compiler_params=pltpu.CompilerParams(use_tc_tiling_on_sc=False),
    )
    def agg(y, src, dst, out, sidx, didx, rows0, rows1, acc_sh,
            sem_i, sem_g):
        c = lax.axis_index("c")
        s = lax.axis_index("s")
        wid = s * NC + c

        di0 = pltpu.async_copy(src.at[pl.ds(wid * CPW, CPW)], sidx, sem_i)
        di1 = pltpu.async_copy(dst.at[pl.ds(wid * CPW, CPW)], didx, sem_i)

        zero16 = _zero16()

        def zbody(i, _):
            for j in range(D // 16):
                rows0[i, pl.ds(j * 16, 16)] = zero16
            return 0
        lax.fori_loop(0, CH, zbody, 0)
        for q in range(RPT // CH):
            pltpu.sync_copy(rows0, acc_sh.at[pl.ds(s * RPT + q * CH, CH)])
        di0.wait()
        di1.wait()
        plsc.subcore_barrier()

        pltpu.async_copy(y.at[sidx.at[0]], rows0, sem_g)

        def body(t, _):
            t0 = 2 * t
            pltpu.make_async_copy(y.at[sidx.at[t0]], rows0, sem_g).wait()
            pltpu.async_copy(y.at[sidx.at[t0 + 1]], rows1, sem_g)
            pltpu.sync_copy(rows0, acc_sh.at[didx.at[t0]], add=True)
            pltpu.make_async_copy(y.at[sidx.at[t0 + 1]], rows1, sem_g).wait()

            @pl.when(t < CPW // 2 - 1)
            def _():
                pltpu.async_copy(y.at[sidx.at[t0 + 2]], rows0, sem_g)
            pltpu.sync_copy(rows1, acc_sh.at[didx.at[t0 + 1]], add=True)
            return 0
        lax.fori_loop(0, CPW // 2, body, 0)
        plsc.subcore_barrier()

        for q in range(RPT // CH):
            off = s * RPT + q * CH
            pltpu.sync_copy(acc_sh.at[pl.ds(off, CH)], rows0)
            pltpu.sync_copy(rows0, out.at[c, pl.ds(off, CH)])

    return agg


_agg_h = _make_agg(HIDDEN)
_agg_o = _make_agg(OUT)


def _degsum_body(dp_ref, out_ref):
    dg = jnp.sum(dp_ref[...], axis=0)
    out_ref[...] = jnp.where(dg > 0, lax.rsqrt(jnp.maximum(dg, 1.0)), 0.0)


def _norms_tc(degparts):
    return pl.pallas_call(
        _degsum_body,
        grid=(NPAD // 128,),
        in_specs=[pl.BlockSpec((NW, 6, 128), lambda i: (0, 0, i))],
        out_specs=pl.BlockSpec((6, 128), lambda i: (0, i)),
        out_shape=jax.ShapeDtypeStruct((6, NPAD), jnp.float32),
    )(degparts)


def _proj_body(x_ref, n_ref, w_ref, o_ref):
    o_ref[...] = jnp.dot(x_ref[...] * n_ref[...], w_ref[...],
                         preferred_element_type=jnp.float32)


def _proj_tc(x, ncol, W):
    H = W.shape[1]
    return pl.pallas_call(
        _proj_body,
        grid=(NPAD // 128,),
        in_specs=[
            pl.BlockSpec((128, 128), lambda i: (i, 0)),
            pl.BlockSpec((128, 1), lambda i: (i, 0)),
            pl.BlockSpec((128, H), lambda i: (0, 0)),
        ],
        out_specs=pl.BlockSpec((128, H), lambda i: (i, 0)),
        out_shape=jax.ShapeDtypeStruct((NPAD, H), jnp.float32),
    )(x, ncol, W)


def _comb2_body(relu, a_ref, c_ref, na_ref, nc_ref, ba_ref, bc_ref, o_ref):
    v = ((a_ref[0] + a_ref[1]) * na_ref[...]
         + (c_ref[0] + c_ref[1]) * nc_ref[...]
         + ba_ref[...] + bc_ref[...])
    o_ref[...] = jnp.maximum(v, 0.0) if relu else v


def _comb2_tc(agg_a, agg_c, n_a, n_c, b_a, b_c, relu):
    D = agg_a.shape[-1]
    return pl.pallas_call(
        functools.partial(_comb2_body, relu),
        grid=(NPAD // 128,),
        in_specs=[
            pl.BlockSpec((NC, 128, D), lambda i: (0, i, 0)),
            pl.BlockSpec((NC, 128, D), lambda i: (0, i, 0)),
            pl.BlockSpec((128, 1), lambda i: (i, 0)),
            pl.BlockSpec((128, 1), lambda i: (i, 0)),
            pl.BlockSpec((1, D), lambda i: (0, 0)),
            pl.BlockSpec((1, D), lambda i: (0, 0)),
        ],
        out_specs=pl.BlockSpec((128, D), lambda i: (i, 0)),
        out_shape=jax.ShapeDtypeStruct((NPAD, D), jnp.float32),
    )(agg_a, agg_c, n_a, n_c, b_a.reshape(1, D), b_c.reshape(1, D))


def _comb1_body(relu, a_ref, na_ref, ba_ref, o_ref):
    v = (a_ref[0] + a_ref[1]) * na_ref[...] + ba_ref[...]
    o_ref[...] = jnp.maximum(v, 0.0) if relu else v


def _comb1_tc(agg_a, n_a, b_a, relu):
    D = agg_a.shape[-1]
    return pl.pallas_call(
        functools.partial(_comb1_body, relu),
        grid=(NPAD // 128,),
        in_specs=[
            pl.BlockSpec((NC, 128, D), lambda i: (0, i, 0)),
            pl.BlockSpec((128, 1), lambda i: (i, 0)),
            pl.BlockSpec((1, D), lambda i: (0, 0)),
        ],
        out_specs=pl.BlockSpec((128, D), lambda i: (i, 0)),
        out_shape=jax.ShapeDtypeStruct((NPAD, D), jnp.float32),
    )(agg_a, n_a, b_a.reshape(1, D))


def kernel(x_paper, x_author, edge_writes, edge_cites, edge_written_by,
           W1_writes, b1_writes, W1_cites, b1_cites, W1_written_by, b1_written_by,
           W2_writes, b2_writes, W2_cites, b2_cites, W2_written_by, b2_written_by):
    pad = ((0, NPAD - N), (0, 0))
    xp = jnp.pad(x_paper, pad)
    xa = jnp.pad(x_author, pad)

    trash = (N + (jnp.arange(E_PAD - E, dtype=jnp.int32) % (NPAD - N)))[None, :]

    def _epad(e):
        return jnp.concatenate([e, jnp.broadcast_to(trash, (2, E_PAD - E))], axis=1)

    ew = _epad(edge_writes)
    ec = _epad(edge_cites)
    eb = _epad(edge_written_by)
    ew_s, ew_d = ew[0], ew[1]
    ec_s, ec_d = ec[0], ec[1]
    eb_s, eb_d = eb[0], eb[1]
    ew_s2, ew_d2 = ew_s.reshape(-1, CH), ew_d.reshape(-1, CH)
    ec_s2, ec_d2 = ec_s.reshape(-1, CH), ec_d.reshape(-1, CH)
    eb_s2, eb_d2 = eb_s.reshape(-1, CH), eb_d.reshape(-1, CH)

    degparts = _degrees_sc(ew_s, ew_d, ec_s, ec_d, eb_s, eb_d)
    norms = _norms_tc(degparts)
    nsw = norms[0].reshape(NPAD, 1)
    ndw = norms[1].reshape(NPAD, 1)
    nsc = norms[2].reshape(NPAD, 1)
    ndc = norms[3].reshape(NPAD, 1)
    nsb = norms[4].reshape(NPAD, 1)
    ndb = norms[5].reshape(NPAD, 1)

    y1a = _proj_tc(xa, nsw, W1_writes)
    y1c = _proj_tc(xp, nsc, W1_cites)
    y1b = _proj_tc(xp, nsb, W1_written_by)
    s1w = _agg_h(y1a, ew_s2, ew_d2)
    s1c = _agg_h(y1c, ec_s2, ec_d2)
    s1b = _agg_h(y1b, eb_s2, eb_d2)
    h_paper = _comb2_tc(s1w, s1c, ndw, ndc, b1_writes, b1_cites, relu=True)
    h_author = _comb1_tc(s1b, ndb, b1_written_by, relu=True)

    y2a = _proj_tc(h_author, nsw, W2_writes)
    y2c = _proj_tc(h_paper, nsc, W2_cites)
    y2b = _proj_tc(h_paper, nsb, W2_written_by)
    s2w = _agg_o(y2a, ew_s2, ew_d2)
    s2c = _agg_o(y2c, ec_s2, ec_d2)
    s2b = _agg_o(y2b, eb_s2, eb_d2)
    out_paper = _comb2_tc(s2w, s2c, ndw, ndc, b2_writes, b2_cites, relu=False)
    out_author = _comb1_tc(s2b, ndb, b2_written_by, relu=False)

    return out_paper[:N], out_author[:N]


# speedup vs baseline: 3.2685x; 1.4360x over previous
import functools

import jax
import jax.numpy as jnp
from jax import lax
from jax.experimental import pallas as pl
from jax.experimental.pallas import tpu as pltpu
from jax.experimental.pallas import tpu_sc as plsc

N = 10000
NPAD = 10240
D_IN = 128
HIDDEN = 128
OUT = 64
E = 160000
CH = 128
NC = 2
NS = 16
NW = NC * NS
E_PAD = 163840
CPW = E_PAD // (NW * CH)
EPW = CPW * CH
RPT = NPAD // NS


def _zero16():
    return jnp.zeros((16,), jnp.float32)

def _worker_id():
    return lax.axis_index("s") * NC + lax.axis_index("c")


@functools.partial(
    pl.kernel,
    out_type=jax.ShapeDtypeStruct((NW, 6, NPAD), jnp.float32),
    mesh=plsc.VectorSubcoreMesh(core_axis_name="c", subcore_axis_name="s"),
    scratch_types=[
        [pltpu.VMEM((EPW,), jnp.int32) for _ in range(6)],
        [pltpu.VMEM((NPAD,), jnp.float32) for _ in range(6)],
        pltpu.SemaphoreType.DMA,
    ],
    compiler_params=pltpu.CompilerParams(needs_layout_passes=False),
)
def _degrees_sc(e0, e1, e2, e3, e4, e5, out, idxs, hists, sem):
    wid = _worker_id()
    base = wid * EPW

    copies = [
        pltpu.async_copy(arr.at[pl.ds(base, EPW)], idx_v, sem)
        for arr, idx_v in zip((e0, e1, e2, e3, e4, e5), idxs)
    ]

    zero16 = _zero16()
    one16 = jnp.ones((16,), jnp.float32)

    def zbody(i, _):
        for h in hists:
            h[pl.ds(i * 16, 16)] = zero16
        return 0
    lax.fori_loop(0, NPAD // 16, zbody, 0)
    for cp in copies:
        cp.wait()

    for idx_v, hist in zip(idxs, hists):
        def body(i, _, idx_v=idx_v, hist=hist):
            idx16 = idx_v[pl.ds(i * 16, 16)]
            plsc.addupdate_scatter(hist, [idx16], one16)
            return 0
        lax.fori_loop(0, EPW // 16, body, 0)

    for r, hist in enumerate(hists):
        pltpu.sync_copy(hist, out.at[wid, r])


def _make_agg(D):
    @functools.partial(
        pl.kernel,
        out_type=jax.ShapeDtypeStruct((NC, NPAD, D), jnp.float32),
        mesh=plsc.VectorSubcoreMesh(core_axis_name="c", subcore_axis_name="s"),
        scratch_types=[
            pltpu.VMEM((CPW, CH), jnp.int32),
            pltpu.VMEM((CPW, CH), jnp.int32),
            pltpu.VMEM((CH, D), jnp.float32),
            pltpu.VMEM((CH, D), jnp.float32),
            pltpu.VMEM_SHARED((NPAD, D), jnp.float32),
            pltpu.SemaphoreType.DMA,
            pltpu.SemaphoreType.DMA,
        ],
        compiler_params=pltpu.CompilerParams(use_tc_tiling_on_sc=False),
    )
    def agg(y, src, dst, out, sidx, didx, rows0, rows1, acc_sh,
            sem_i, sem_g):
        c = lax.axis_index("c")
        s = lax.axis_index("s")
        wid = s * NC + c

        di0 = pltpu.async_copy(src.at[pl.ds(wid * CPW, CPW)], sidx, sem_i)
        di1 = pltpu.async_copy(dst.at[pl.ds(wid * CPW, CPW)], didx, sem_i)

        zero16 = _zero16()

        def zbody(i, _):
            for j in range(D // 16):
                rows0[i, pl.ds(j * 16, 16)] = zero16
            return 0
        lax.fori_loop(0, CH, zbody, 0)
        for q in range(RPT // CH):
            pltpu.sync_copy(rows0, acc_sh.at[pl.ds(s * RPT + q * CH, CH)])
        di0.wait()
        di1.wait()
        plsc.subcore_barrier()

        pltpu.async_copy(y.at[sidx.at[0]], rows0, sem_g)

        def body(t, _):
            t0 = 2 * t
            pltpu.make_async_copy(y.at[sidx.at[t0]], rows0, sem_g).wait()
            pltpu.async_copy(y.at[sidx.at[t0 + 1]], rows1, sem_g)
            pltpu.sync_copy(rows0, acc_sh.at[didx.at[t0]], add=True)
            pltpu.make_async_copy(y.at[sidx.at[t0 + 1]], rows1, sem_g).wait()

            @pl.when(t < CPW // 2 - 1)
            def _():
                pltpu.async_copy(y.at[sidx.at[t0 + 2]], rows0, sem_g)
            pltpu.sync_copy(rows1, acc_sh.at[didx.at[t0 + 1]], add=True)
            return 0
        lax.fori_loop(0, CPW // 2, body, 0)
        plsc.subcore_barrier()

        for q in range(RPT // CH):
            off = s * RPT + q * CH
            pltpu.sync_copy(acc_sh.at[pl.ds(off, CH)], rows0)
            pltpu.sync_copy(rows0, out.at[c, pl.ds(off, CH)])

    return agg


_agg_h = _make_agg(HIDDEN)
_agg_o = _make_agg(OUT)


def _degsum_body(dp_ref, out_ref):
    dg = jnp.sum(dp_ref[...], axis=0)
    out_ref[...] = jnp.where(dg > 0, lax.rsqrt(jnp.maximum(dg, 1.0)), 0.0)


_BR = 1024


def _norms_tc(degparts):
    return pl.pallas_call(
        _degsum_body,
        grid=(NPAD // _BR,),
        in_specs=[pl.BlockSpec((NW, 6, _BR), lambda i: (0, 0, i))],
        out_specs=pl.BlockSpec((6, _BR), lambda i: (0, i)),
        out_shape=jax.ShapeDtypeStruct((6, NPAD), jnp.float32),
    )(degparts)


def _proj_body(x_ref, n_ref, w_ref, o_ref):
    o_ref[...] = jnp.dot(x_ref[...] * n_ref[...], w_ref[...],
                         preferred_element_type=jnp.float32)


def _proj_tc(x, ncol, W):
    H = W.shape[1]
    return pl.pallas_call(
        _proj_body,
        grid=(NPAD // _BR,),
        in_specs=[
            pl.BlockSpec((_BR, 128), lambda i: (i, 0)),
            pl.BlockSpec((_BR, 1), lambda i: (i, 0)),
            pl.BlockSpec((128, H), lambda i: (0, 0)),
        ],
        out_specs=pl.BlockSpec((_BR, H), lambda i: (i, 0)),
        out_shape=jax.ShapeDtypeStruct((NPAD, H), jnp.float32),
    )(x, ncol, W)


def _comb2_body(relu, a_ref, c_ref, na_ref, nc_ref, ba_ref, bc_ref, o_ref):
    v = ((a_ref[0] + a_ref[1]) * na_ref[...]
         + (c_ref[0] + c_ref[1]) * nc_ref[...]
         + ba_ref[...] + bc_ref[...])
    o_ref[...] = jnp.maximum(v, 0.0) if relu else v


def _comb2_tc(agg_a, agg_c, n_a, n_c, b_a, b_c, relu):
    D = agg_a.shape[-1]
    return pl.pallas_call(
        functools.partial(_comb2_body, relu),
        grid=(NPAD // _BR,),
        in_specs=[
            pl.BlockSpec((NC, _BR, D), lambda i: (0, i, 0)),
            pl.BlockSpec((NC, _BR, D), lambda i: (0, i, 0)),
            pl.BlockSpec((_BR, 1), lambda i: (i, 0)),
            pl.BlockSpec((_BR, 1), lambda i: (i, 0)),
            pl.BlockSpec((1, D), lambda i: (0, 0)),
            pl.BlockSpec((1, D), lambda i: (0, 0)),
        ],
        out_specs=pl.BlockSpec((_BR, D), lambda i: (i, 0)),
        out_shape=jax.ShapeDtypeStruct((NPAD, D), jnp.float32),
    )(agg_a, agg_c, n_a, n_c, b_a.reshape(1, D), b_c.reshape(1, D))


def _comb1_body(relu, a_ref, na_ref, ba_ref, o_ref):
    v = (a_ref[0] + a_ref[1]) * na_ref[...] + ba_ref[...]
    o_ref[...] = jnp.maximum(v, 0.0) if relu else v


def _comb1_tc(agg_a, n_a, b_a, relu):
    D = agg_a.shape[-1]
    return pl.pallas_call(
        functools.partial(_comb1_body, relu),
        grid=(NPAD // _BR,),
        in_specs=[
            pl.BlockSpec((NC, _BR, D), lambda i: (0, i, 0)),
            pl.BlockSpec((_BR, 1), lambda i: (i, 0)),
            pl.BlockSpec((1, D), lambda i: (0, 0)),
        ],
        out_specs=pl.BlockSpec((_BR, D), lambda i: (i, 0)),
        out_shape=jax.ShapeDtypeStruct((NPAD, D), jnp.float32),
    )(agg_a, n_a, b_a.reshape(1, D))


def kernel(x_paper, x_author, edge_writes, edge_cites, edge_written_by,
           W1_writes, b1_writes, W1_cites, b1_cites, W1_written_by, b1_written_by,
           W2_writes, b2_writes, W2_cites, b2_cites, W2_written_by, b2_written_by):
    pad = ((0, NPAD - N), (0, 0))
    xp = jnp.pad(x_paper, pad)
    xa = jnp.pad(x_author, pad)

    trash = (N + (jnp.arange(E_PAD - E, dtype=jnp.int32) % (NPAD - N)))[None, :]

    def _epad(e):
        return jnp.concatenate([e, jnp.broadcast_to(trash, (2, E_PAD - E))], axis=1)

    ew = _epad(edge_writes)
    ec = _epad(edge_cites)
    eb = _epad(edge_written_by)
    ew_s, ew_d = ew[0], ew[1]
    ec_s, ec_d = ec[0], ec[1]
    eb_s, eb_d = eb[0], eb[1]
    ew_s2, ew_d2 = ew_s.reshape(-1, CH), ew_d.reshape(-1, CH)
    ec_s2, ec_d2 = ec_s.reshape(-1, CH), ec_d.reshape(-1, CH)
    eb_s2, eb_d2 = eb_s.reshape(-1, CH), eb_d.reshape(-1, CH)

    degparts = _degrees_sc(ew_s, ew_d, ec_s, ec_d, eb_s, eb_d)
    norms = _norms_tc(degparts)
    nsw = norms[0].reshape(NPAD, 1)
    ndw = norms[1].reshape(NPAD, 1)
    nsc = norms[2].reshape(NPAD, 1)
    ndc = norms[3].reshape(NPAD, 1)
    nsb = norms[4].reshape(NPAD, 1)
    ndb = norms[5].reshape(NPAD, 1)

    y1a = _proj_tc(xa, nsw, W1_writes)
    y1c = _proj_tc(xp, nsc, W1_cites)
    y1b = _proj_tc(xp, nsb, W1_written_by)
    s1w = _agg_h(y1a, ew_s2, ew_d2)
    s1c = _agg_h(y1c, ec_s2, ec_d2)
    s1b = _agg_h(y1b, eb_s2, eb_d2)
    h_paper = _comb2_tc(s1w, s1c, ndw, ndc, b1_writes, b1_cites, relu=True)
    h_author = _comb1_tc(s1b, ndb, b1_written_by, relu=True)

    y2a = _proj_tc(h_author, nsw, W2_writes)
    y2c = _proj_tc(h_paper, nsc, W2_cites)
    y2b = _proj_tc(h_paper, nsb, W2_written_by)
    s2w = _agg_o(y2a, ew_s2, ew_d2)
    s2c = _agg_o(y2c, ec_s2, ec_d2)
    s2b = _agg_o(y2b, eb_s2, eb_d2)
    out_paper = _comb2_tc(s2w, s2c, ndw, ndc, b2_writes, b2_cites, relu=False)
    out_author = _comb1_tc(s2b, ndb, b2_written_by, relu=False)

    return out_paper[:N], out_author[:N]


# speedup vs baseline: 3.2797x; 1.0034x over previous
import functools

import jax
import jax.numpy as jnp
from jax import lax
from jax.experimental import pallas as pl
from jax.experimental.pallas import tpu as pltpu
from jax.experimental.pallas import tpu_sc as plsc

N = 10000
NPAD = 10240
D_IN = 128
HIDDEN = 128
OUT = 64
E = 160000
CH = 128
NC = 2
NS = 16
NW = NC * NS
E_PAD = 163840
CPW = E_PAD // (NW * CH)
EPW = CPW * CH
RPT = NPAD // NS


def _zero16():
    return jnp.zeros((16,), jnp.float32)

def _worker_id():
    return lax.axis_index("s") * NC + lax.axis_index("c")


@functools.partial(
    pl.kernel,
    out_type=jax.ShapeDtypeStruct((NW, 6, NPAD), jnp.float32),
    mesh=plsc.VectorSubcoreMesh(core_axis_name="c", subcore_axis_name="s"),
    scratch_types=[
        [pltpu.VMEM((EPW,), jnp.int32) for _ in range(6)],
        [pltpu.VMEM((NPAD,), jnp.float32) for _ in range(6)],
        pltpu.SemaphoreType.DMA,
    ],
    compiler_params=pltpu.CompilerParams(needs_layout_passes=False),
)
def _degrees_sc(e0, e1, e2, e3, e4, e5, out, idxs, hists, sem):
    wid = _worker_id()
    base = wid * EPW

    copies = [
        pltpu.async_copy(arr.at[pl.ds(base, EPW)], idx_v, sem)
        for arr, idx_v in zip((e0, e1, e2, e3, e4, e5), idxs)
    ]

    zero16 = _zero16()
    one16 = jnp.ones((16,), jnp.float32)

    def zbody(i, _):
        for h in hists:
            h[pl.ds(i * 16, 16)] = zero16
        return 0
    lax.fori_loop(0, NPAD // 16, zbody, 0)
    for cp in copies:
        cp.wait()

    for idx_v, hist in zip(idxs, hists):
        def body(i, _, idx_v=idx_v, hist=hist):
            idx16 = idx_v[pl.ds(i * 16, 16)]
            plsc.addupdate_scatter(hist, [idx16], one16)
            return 0
        lax.fori_loop(0, EPW // 16, body, 0)

    for r, hist in enumerate(hists):
        pltpu.sync_copy(hist, out.at[wid, r])


def _make_agg(D, tc_tiling):
    @functools.partial(
        pl.kernel,
        out_type=jax.ShapeDtypeStruct((NC, NPAD, D), jnp.float32),
        mesh=plsc.VectorSubcoreMesh(core_axis_name="c", subcore_axis_name="s"),
        scratch_types=[
            pltpu.VMEM((CPW, CH), jnp.int32),
            pltpu.VMEM((CPW, CH), jnp.int32),
            pltpu.VMEM((CH, D), jnp.float32),
            pltpu.VMEM((CH, D), jnp.float32),
            pltpu.VMEM_SHARED((NPAD, D), jnp.float32),
            pltpu.SemaphoreType.DMA,
            pltpu.SemaphoreType.DMA,
        ],
        compiler_params=pltpu.CompilerParams(use_tc_tiling_on_sc=tc_tiling),
    )
    def agg(y, src, dst, out, sidx, didx, rows0, rows1, acc_sh,
            sem_i, sem_g):
        c = lax.axis_index("c")
        s = lax.axis_index("s")
        wid = s * NC + c

        di0 = pltpu.async_copy(src.at[pl.ds(wid * CPW, CPW)], sidx, sem_i)
        di1 = pltpu.async_copy(dst.at[pl.ds(wid * CPW, CPW)], didx, sem_i)

        zero16 = _zero16()

        def zbody(i, _):
            for j in range(D // 16):
                rows0[i, pl.ds(j * 16, 16)] = zero16
            return 0
        lax.fori_loop(0, CH, zbody, 0)
        for q in range(RPT // CH):
            pltpu.sync_copy(rows0, acc_sh.at[pl.ds(s * RPT + q * CH, CH)])
        di0.wait()
        di1.wait()
        plsc.subcore_barrier()

        pltpu.async_copy(y.at[sidx.at[0]], rows0, sem_g)

        def body(t, _):
            t0 = 2 * t
            pltpu.make_async_copy(y.at[sidx.at[t0]], rows0, sem_g).wait()
            pltpu.async_copy(y.at[sidx.at[t0 + 1]], rows1, sem_g)
            pltpu.sync_copy(rows0, acc_sh.at[didx.at[t0]], add=True)
            pltpu.make_async_copy(y.at[sidx.at[t0 + 1]], rows1, sem_g).wait()

            @pl.when(t < CPW // 2 - 1)
            def _():
                pltpu.async_copy(y.at[sidx.at[t0 + 2]], rows0, sem_g)
            pltpu.sync_copy(rows1, acc_sh.at[didx.at[t0 + 1]], add=True)
            return 0
        lax.fori_loop(0, CPW // 2, body, 0)
        plsc.subcore_barrier()

        for q in range(RPT // CH):
            off = s * RPT + q * CH
            pltpu.sync_copy(acc_sh.at[pl.ds(off, CH)], rows0)
            pltpu.sync_copy(rows0, out.at[c, pl.ds(off, CH)])

    return agg


_agg_h = _make_agg(HIDDEN, tc_tiling=True)
_agg_o = _make_agg(OUT, tc_tiling=False)


def _degsum_body(dp_ref, out_ref):
    dg = jnp.sum(dp_ref[...], axis=0)
    out_ref[...] = jnp.where(dg > 0, lax.rsqrt(jnp.maximum(dg, 1.0)), 0.0)


_BR = 1024


def _norms_tc(degparts):
    return pl.pallas_call(
        _degsum_body,
        grid=(NPAD // _BR,),
        in_specs=[pl.BlockSpec((NW, 6, _BR), lambda i: (0, 0, i))],
        out_specs=pl.BlockSpec((6, _BR), lambda i: (0, i)),
        out_shape=jax.ShapeDtypeStruct((6, NPAD), jnp.float32),
    )(degparts)


def _proj_body(x_ref, n_ref, w_ref, o_ref):
    o_ref[...] = jnp.dot(x_ref[...] * n_ref[...], w_ref[...],
                         preferred_element_type=jnp.float32)


def _proj_tc(x, ncol, W):
    H = W.shape[1]
    return pl.pallas_call(
        _proj_body,
        grid=(NPAD // _BR,),
        in_specs=[
            pl.BlockSpec((_BR, 128), lambda i: (i, 0)),
            pl.BlockSpec((_BR, 1), lambda i: (i, 0)),
            pl.BlockSpec((128, H), lambda i: (0, 0)),
        ],
        out_specs=pl.BlockSpec((_BR, H), lambda i: (i, 0)),
        out_shape=jax.ShapeDtypeStruct((NPAD, H), jnp.float32),
    )(x, ncol, W)


def _comb2_body(relu, a_ref, c_ref, na_ref, nc_ref, ba_ref, bc_ref, o_ref):
    v = ((a_ref[0] + a_ref[1]) * na_ref[...]
         + (c_ref[0] + c_ref[1]) * nc_ref[...]
         + ba_ref[...] + bc_ref[...])
    o_ref[...] = jnp.maximum(v, 0.0) if relu else v


def _comb2_tc(agg_a, agg_c, n_a, n_c, b_a, b_c, relu):
    D = agg_a.shape[-1]
    return pl.pallas_call(
        functools.partial(_comb2_body, relu),
        grid=(NPAD // _BR,),
        in_specs=[
            pl.BlockSpec((NC, _BR, D), lambda i: (0, i, 0)),
            pl.BlockSpec((NC, _BR, D), lambda i: (0, i, 0)),
            pl.BlockSpec((_BR, 1), lambda i: (i, 0)),
            pl.BlockSpec((_BR, 1), lambda i: (i, 0)),
            pl.BlockSpec((1, D), lambda i: (0, 0)),
            pl.BlockSpec((1, D), lambda i: (0, 0)),
        ],
        out_specs=pl.BlockSpec((_BR, D), lambda i: (i, 0)),
        out_shape=jax.ShapeDtypeStruct((NPAD, D), jnp.float32),
    )(agg_a, agg_c, n_a, n_c, b_a.reshape(1, D), b_c.reshape(1, D))


def _comb1_body(relu, a_ref, na_ref, ba_ref, o_ref):
    v = (a_ref[0] + a_ref[1]) * na_ref[...] + ba_ref[...]
    o_ref[...] = jnp.maximum(v, 0.0) if relu else v


def _comb1_tc(agg_a, n_a, b_a, relu):
    D = agg_a.shape[-1]
    return pl.pallas_call(
        functools.partial(_comb1_body, relu),
        grid=(NPAD // _BR,),
        in_specs=[
            pl.BlockSpec((NC, _BR, D), lambda i: (0, i, 0)),
            pl.BlockSpec((_BR, 1), lambda i: (i, 0)),
            pl.BlockSpec((1, D), lambda i: (0, 0)),
        ],
        out_specs=pl.BlockSpec((_BR, D), lambda i: (i, 0)),
        out_shape=jax.ShapeDtypeStruct((NPAD, D), jnp.float32),
    )(agg_a, n_a, b_a.reshape(1, D))


def kernel(x_paper, x_author, edge_writes, edge_cites, edge_written_by,
           W1_writes, b1_writes, W1_cites, b1_cites, W1_written_by, b1_written_by,
           W2_writes, b2_writes, W2_cites, b2_cites, W2_written_by, b2_written_by):
    pad = ((0, NPAD - N), (0, 0))
    xp = jnp.pad(x_paper, pad)
    xa = jnp.pad(x_author, pad)

    trash = (N + (jnp.arange(E_PAD - E, dtype=jnp.int32) % (NPAD - N)))[None, :]

    def _epad(e):
        return jnp.concatenate([e, jnp.broadcast_to(trash, (2, E_PAD - E))], axis=1)

    ew = _epad(edge_writes)
    ec = _epad(edge_cites)
    eb = _epad(edge_written_by)
    ew_s, ew_d = ew[0], ew[1]
    ec_s, ec_d = ec[0], ec[1]
    eb_s, eb_d = eb[0], eb[1]
    ew_s2, ew_d2 = ew_s.reshape(-1, CH), ew_d.reshape(-1, CH)
    ec_s2, ec_d2 = ec_s.reshape(-1, CH), ec_d.reshape(-1, CH)
    eb_s2, eb_d2 = eb_s.reshape(-1, CH), eb_d.reshape(-1, CH)

    degparts = _degrees_sc(ew_s, ew_d, ec_s, ec_d, eb_s, eb_d)
    norms = _norms_tc(degparts)
    nsw = norms[0].reshape(NPAD, 1)
    ndw = norms[1].reshape(NPAD, 1)
    nsc = norms[2].reshape(NPAD, 1)
    ndc = norms[3].reshape(NPAD, 1)
    nsb = norms[4].reshape(NPAD, 1)
    ndb = norms[5].reshape(NPAD, 1)

    y1a = _proj_tc(xa, nsw, W1_writes)
    y1c = _proj_tc(xp, nsc, W1_cites)
    y1b = _proj_tc(xp, nsb, W1_written_by)
    s1w = _agg_h(y1a, ew_s2, ew_d2)
    s1c = _agg_h(y1c, ec_s2, ec_d2)
    s1b = _agg_h(y1b, eb_s2, eb_d2)
    h_paper = _comb2_tc(s1w, s1c, ndw, ndc, b1_writes, b1_cites, relu=True)
    h_author = _comb1_tc(s1b, ndb, b1_written_by, relu=True)

    y2a = _proj_tc(h_author, nsw, W2_writes)
    y2c = _proj_tc(h_paper, nsc, W2_cites)
    y2b = _proj_tc(h_paper, nsb, W2_written_by)
    s2w = _agg_o(y2a, ew_s2, ew_d2)
    s2c = _agg_o(y2c, ec_s2, ec_d2)
    s2b = _agg_o(y2b, eb_s2, eb_d2)
    out_paper = _comb2_tc(s2w, s2c, ndw, ndc, b2_writes, b2_cites, relu=False)
    out_author = _comb1_tc(s2b, ndb, b2_written_by, relu=False)

    return out_paper[:N], out_author[:N]
